# baseline (speedup 1.0000x reference)
import functools

import jax
import jax.numpy as jnp
from jax import lax
from jax.experimental import pallas as pl
from jax.experimental.pallas import tpu as pltpu
import jax.experimental.pallas.tpu_sc as plsc


def _buckets(p, S):
    f = jnp.floor(p)
    r0 = jnp.clip(f - 1.0, 0.0, S - 4.0)
    start = (p - 1.0) - 0.5
    ylfs, yhfs, lys, hys, vs = [], [], [], [], []
    for j in range(3):
        q = start + (j + 0.5)
        v = (q > -1.0) & (q < S)
        y = jnp.maximum(q, 0.0)
        ylf = jnp.floor(y)
        over = ylf >= S - 1.0
        ylf = jnp.where(over, S - 1.0, ylf)
        yhf = jnp.where(over, S - 1.0, ylf + 1.0)
        ly = jnp.where(over, 0.0, y - ylf)
        hy = 1.0 - ly
        ylfs.append(ylf); yhfs.append(yhf); lys.append(ly); hys.append(hy)
        vs.append(v.astype(p.dtype))
    cols = []
    for k in range(4):
        r = r0 + k
        acc = jnp.zeros_like(p)
        for j in range(3):
            acc = acc + vs[j] * (hys[j] * (ylfs[j] == r).astype(p.dtype)
                                 + lys[j] * (yhfs[j] == r).astype(p.dtype))
        cols.append(acc)
    return r0, cols


def _prep_body(x_ref, w_ref, pos_ref, idx_ref, wts_ref, flg_ref, *,
               bn, n_per_b, S):
    i = pl.program_id(0)
    x = x_ref[...]
    w = w_ref[...]
    pos_t = lax.dot_general(w, x, (((0,), (1,)), ((), ())),
                            preferred_element_type=jnp.float32)
    ii2 = lax.broadcasted_iota(jnp.int32, (2, 2), 0)
    jj2 = lax.broadcasted_iota(jnp.int32, (2, 2), 1)
    eye2 = (ii2 == jj2).astype(jnp.float32)
    pos_ref[...] = lax.dot_general(pos_t, eye2, (((0,), (0,)), ((), ())),
                                   preferred_element_type=jnp.float32)
    px = pos_t[0:1, :] * float(S)
    py = pos_t[1:2, :] * float(S)
    cx0, wx = _buckets(px, S)
    ry0, wy = _buckets(py, S)
    rows = lax.broadcasted_iota(jnp.int32, (1, bn), 1) + i * bn
    bvec = (rows // n_per_b).astype(jnp.float32)
    base = bvec * float(S * S) + ry0 * float(S) + cx0
    idx_t = jnp.concatenate(
        [base + float(a * S + k) for a in range(4) for k in range(4)],
        axis=0)
    wts_t = jnp.concatenate(
        [wy[a] * wx[k] * (1.0 / 9.0) for a in range(4) for k in range(4)],
        axis=0)
    ii = lax.broadcasted_iota(jnp.int32, (16, 16), 0)
    jj = lax.broadcasted_iota(jnp.int32, (16, 16), 1)
    eye16 = (ii == jj).astype(jnp.float32)
    wts = lax.dot_general(wts_t, eye16, (((0,), (0,)), ((), ())),
                          preferred_element_type=jnp.float32)
    wts_ref[...] = wts
    idx = lax.dot_general(idx_t, eye16, (((0,), (0,)), ((), ())),
                          preferred_element_type=jnp.float32)
    idx_ref[...] = idx.astype(jnp.int32)
    aw3 = jnp.sum(jnp.abs(wts).reshape(bn // 8, 8, 16), axis=2)
    nz8 = (aw3 > 0.0).astype(jnp.float32)
    any8 = (jnp.sum(aw3, axis=1, keepdims=True) > 0.0).astype(jnp.float32)
    flg_ref[...] = jnp.concatenate(
        [any8, nz8, jnp.zeros((bn // 8, 7), jnp.float32)], axis=1)


def _prep(xf, W, n_per_b, S, bn=1600):
    nn = xf.shape[0]
    c = xf.shape[1]
    grid = nn // bn
    return pl.pallas_call(
        functools.partial(_prep_body, bn=bn, n_per_b=n_per_b, S=S),
        grid=(grid,),
        in_specs=[
            pl.BlockSpec((bn, c), lambda i: (i, 0)),
            pl.BlockSpec((c, 2), lambda i: (0, 0)),
        ],
        out_specs=[
            pl.BlockSpec((bn, 2), lambda i: (i, 0)),
            pl.BlockSpec((bn, 16), lambda i: (i, 0)),
            pl.BlockSpec((bn, 16), lambda i: (i, 0)),
            pl.BlockSpec((bn // 8, 16), lambda i: (i, 0)),
        ],
        out_shape=[
            jax.ShapeDtypeStruct((nn, 2), jnp.float32),
            jax.ShapeDtypeStruct((nn, 16), jnp.int32),
            jax.ShapeDtypeStruct((nn, 16), jnp.float32),
            jax.ShapeDtypeStruct((nn // 8, 16), jnp.float32),
        ],
    )(xf, W)


def _tr_body(f_ref, t_ref):
    c = f_ref.shape[1]
    ii = lax.broadcasted_iota(jnp.int32, (c, c), 0)
    jj = lax.broadcasted_iota(jnp.int32, (c, c), 1)
    eye = (ii == jj).astype(jnp.float32)
    for yy in range(f_ref.shape[2]):
        x = f_ref[0, :, yy, :]
        t_ref[0, yy] = lax.dot_general(
            x, eye, (((0,), (0,)), ((), ())),
            preferred_element_type=jnp.float32)


def _transpose_table(conv, hblk=8):
    b, c, h, w = conv.shape
    return pl.pallas_call(
        _tr_body,
        grid=(b, h // hblk),
        in_specs=[pl.BlockSpec((1, c, hblk, w), lambda i, j: (i, 0, j, 0))],
        out_specs=pl.BlockSpec((1, hblk, w, c), lambda i, j: (i, j, 0, 0)),
        out_shape=jax.ShapeDtypeStruct((b, h, w, c), jnp.float32),
    )(conv)


_CH = 8
_SUP = 8


def _sc_gather_body(table_h, idx_h, wts_h, flg_h, skip_h, fl_v, idx_v, wts_v,
                    rows_v, out_v, gsem, *, nn, c):
    nw = 32
    wid = lax.axis_index("s") * 2 + lax.axis_index("c")
    nsup = nn // (_CH * _SUP)
    q, r = divmod(nsup, nw)
    start = wid * q + jnp.minimum(wid, r)
    count = q + jnp.where(wid < r, 1, 0)
    cc = c // 16

    pltpu.sync_copy(flg_h.at[pl.ds(start * _SUP, q * _SUP)],
                    fl_v.at[pl.ds(0, q * _SUP)])

    @pl.when(wid < r)
    def _():
        pltpu.sync_copy(flg_h.at[pl.ds((start + q) * _SUP, _SUP)],
                        fl_v.at[pl.ds(q * _SUP, _SUP)])

    def sup(j, carry):
        sid = start + j

        def chunk(c8, carry2):
            cid = sid * _SUP + c8
            base = cid * _CH
            flr = fl_v[j * _SUP + c8, :]
            chunk_nz = flr[0] > 0.0

            @pl.when(chunk_nz)
            def _():
                pltpu.sync_copy(wts_h.at[pl.ds(base, _CH)], wts_v)
                pltpu.sync_copy(idx_h.at[pl.ds(cid * 128, 128)], idx_v)
                pltpu.async_copy(table_h.at[idx_v], rows_v, gsem).wait()
                for i in range(_CH):
                    node_nz = flr[1 + i] > 0.0

                    @pl.when(node_nz)
                    def _(i=i):
                        wrow = wts_v[i, :]

                        def rstep(rr, accs):
                            w_spl = wrow.at[jnp.full((16,), rr, jnp.int32)] \
                                .get(mode="promise_in_bounds")
                            new = []
                            for ch in range(cc):
                                v = rows_v[i * 16 + rr, pl.ds(ch * 16, 16)]
                                new.append(accs[ch] + v * w_spl)
                            return tuple(new)

                        accs = lax.fori_loop(
                            0, 16, rstep,
                            tuple(jnp.zeros((16,), jnp.float32)
                                  for _ in range(cc)))
                        for ch in range(cc):
                            out_v[i, pl.ds(ch * 16, 16)] = accs[ch]

                    @pl.when(jnp.logical_not(node_nz))
                    def _(i=i):
                        for ch in range(cc):
                            out_v[i, pl.ds(ch * 16, 16)] = jnp.zeros(
                                (16,), jnp.float32)

                pltpu.sync_copy(out_v, skip_h.at[pl.ds(base, _CH)])

            return carry2

        lax.fori_loop(0, _SUP, chunk, 0)
        return carry

    lax.fori_loop(0, count, sup, 0)


def _sc_gather(table, idx, wts, flags, nn, c):
    kfn = functools.partial(
        pl.kernel,
        out_type=jax.ShapeDtypeStruct((nn, c), jnp.float32),
        mesh=plsc.VectorSubcoreMesh(core_axis_name="c", subcore_axis_name="s",
                                    num_cores=2, num_subcores=16),
        scratch_types=[
            pltpu.VMEM(((nn // (_CH * _SUP * 32) + 1) * _SUP, 16),
                       jnp.float32),
            pltpu.VMEM((128,), jnp.int32),
            pltpu.VMEM((_CH, 16), jnp.float32),
            pltpu.VMEM((_CH * 16, c), jnp.float32),
            pltpu.VMEM((_CH, c), jnp.float32),
            pltpu.SemaphoreType.DMA,
        ],
        compiler_params=pltpu.CompilerParams(use_tc_tiling_on_sc=True),
    )(functools.partial(_sc_gather_body, nn=nn, c=c))
    return kfn(table, idx, wts, flags)


def _asm_body(x_ref, skip_ref, pos_ref, wts_ref, out_ref, *, c):
    out_ref[:, 0:c] = x_ref[...]
    sel = jnp.sum(jnp.abs(wts_ref[...]), axis=1, keepdims=True) > 0.0
    out_ref[:, c:2 * c] = jnp.where(sel, skip_ref[...], 0.0)
    out_ref[:, 2 * c:2 * c + 2] = pos_ref[...]


def _asm(xf, skip, pos, wts, bn=2000):
    nn, c = xf.shape
    grid = nn // bn
    return pl.pallas_call(
        functools.partial(_asm_body, c=c),
        grid=(grid,),
        in_specs=[
            pl.BlockSpec((bn, c), lambda i: (i, 0)),
            pl.BlockSpec((bn, c), lambda i: (i, 0)),
            pl.BlockSpec((bn, 2), lambda i: (i, 0)),
            pl.BlockSpec((bn, 16), lambda i: (i, 0)),
        ],
        out_specs=pl.BlockSpec((bn, 2 * c + 2), lambda i: (i, 0)),
        out_shape=jax.ShapeDtypeStruct((nn, 2 * c + 2), jnp.float32),
    )(xf, skip, pos, wts)


def kernel(x, adj, conv_layer, W):
    B, N, C = x.shape
    _, _, H, Wd = conv_layer.shape
    S = Wd
    nn = B * N
    xf = x.reshape(nn, C)
    pos_f, idx, wts, flags = _prep(xf, W, n_per_b=N, S=S)
    table = _transpose_table(conv_layer).reshape(B * H * Wd, C)
    skip = _sc_gather(table, idx.reshape(nn * 16), wts, flags, nn, C)
    sel = (jnp.sum(jnp.abs(wts), axis=1, keepdims=True) > 0.0).reshape(B, N, 1)
    pos = pos_f.reshape(B, N, 2)
    out = jnp.concatenate(
        [x, jnp.where(sel, skip.reshape(B, N, C), 0.0), pos], axis=-1)
    return (out, pos)


# speedup vs baseline: 1.3433x; 1.3433x over previous
import functools

import jax
import jax.numpy as jnp
from jax import lax
from jax.experimental import pallas as pl
from jax.experimental.pallas import tpu as pltpu
import jax.experimental.pallas.tpu_sc as plsc


def _buckets(p, S):
    f = jnp.floor(p)
    r0 = jnp.clip(f - 1.0, 0.0, S - 4.0)
    start = (p - 1.0) - 0.5
    ylfs, yhfs, lys, hys, vs = [], [], [], [], []
    for j in range(3):
        q = start + (j + 0.5)
        v = (q > -1.0) & (q < S)
        y = jnp.maximum(q, 0.0)
        ylf = jnp.floor(y)
        over = ylf >= S - 1.0
        ylf = jnp.where(over, S - 1.0, ylf)
        yhf = jnp.where(over, S - 1.0, ylf + 1.0)
        ly = jnp.where(over, 0.0, y - ylf)
        hy = 1.0 - ly
        ylfs.append(ylf); yhfs.append(yhf); lys.append(ly); hys.append(hy)
        vs.append(v.astype(p.dtype))
    cols = []
    for k in range(4):
        r = r0 + k
        acc = jnp.zeros_like(p)
        for j in range(3):
            acc = acc + vs[j] * (hys[j] * (ylfs[j] == r).astype(p.dtype)
                                 + lys[j] * (yhfs[j] == r).astype(p.dtype))
        cols.append(acc)
    return r0, jnp.concatenate(cols, axis=1)


def _prep_body(x_ref, w_ref, pos_ref, idx_ref, wts_ref, flg_ref, *,
               bn, n_per_b, S):
    i = pl.program_id(0)
    x = x_ref[...]
    w = w_ref[...]
    pos = jnp.dot(x, w, preferred_element_type=jnp.float32)
    pos_ref[...] = pos
    p = pos * float(S)
    px = p[:, 0:1]
    py = p[:, 1:2]
    cx0, wx = _buckets(px, S)
    ry0, wy = _buckets(py, S)
    rows = lax.broadcasted_iota(jnp.int32, (bn, 1), 0) + i * bn
    bvec = (rows // n_per_b).astype(jnp.float32)
    base = (bvec * float(S * S) + ry0 * float(S) + cx0).astype(jnp.int32)
    t = lax.broadcasted_iota(jnp.int32, (1, 16), 1)
    offs = (t // 4) * S + (t % 4)
    idx_ref[...] = base + offs
    wts = jnp.concatenate([wy[:, a:a + 1] * wx for a in range(4)], axis=1)
    wts_ref[...] = wts * (1.0 / 9.0)
    aw3 = jnp.sum(jnp.abs(wts).reshape(bn // 8, 8, 16), axis=2)
    nz8 = (aw3 > 0.0).astype(jnp.float32)
    any8 = (jnp.sum(aw3, axis=1, keepdims=True) > 0.0).astype(jnp.float32)
    flg_ref[...] = jnp.concatenate(
        [any8, nz8, jnp.zeros((bn // 8, 7), jnp.float32)], axis=1)


def _prep(xf, W, n_per_b, S, bn=1600):
    nn = xf.shape[0]
    c = xf.shape[1]
    grid = nn // bn
    return pl.pallas_call(
        functools.partial(_prep_body, bn=bn, n_per_b=n_per_b, S=S),
        grid=(grid,),
        in_specs=[
            pl.BlockSpec((bn, c), lambda i: (i, 0)),
            pl.BlockSpec((c, 2), lambda i: (0, 0)),
        ],
        out_specs=[
            pl.BlockSpec((bn, 2), lambda i: (i, 0)),
            pl.BlockSpec((bn, 16), lambda i: (i, 0)),
            pl.BlockSpec((bn, 16), lambda i: (i, 0)),
            pl.BlockSpec((bn // 8, 16), lambda i: (i, 0)),
        ],
        out_shape=[
            jax.ShapeDtypeStruct((nn, 2), jnp.float32),
            jax.ShapeDtypeStruct((nn, 16), jnp.int32),
            jax.ShapeDtypeStruct((nn, 16), jnp.float32),
            jax.ShapeDtypeStruct((nn // 8, 16), jnp.float32),
        ],
    )(xf, W)


def _tr_body(f_ref, t_ref):
    c = f_ref.shape[1]
    ii = lax.broadcasted_iota(jnp.int32, (c, c), 0)
    jj = lax.broadcasted_iota(jnp.int32, (c, c), 1)
    eye = (ii == jj).astype(jnp.float32)
    for yy in range(f_ref.shape[2]):
        x = f_ref[0, :, yy, :]
        t_ref[0, yy] = lax.dot_general(
            x, eye, (((0,), (0,)), ((), ())),
            preferred_element_type=jnp.float32)


def _transpose_table(conv, hblk=8):
    b, c, h, w = conv.shape
    return pl.pallas_call(
        _tr_body,
        grid=(b, h // hblk),
        in_specs=[pl.BlockSpec((1, c, hblk, w), lambda i, j: (i, 0, j, 0))],
        out_specs=pl.BlockSpec((1, hblk, w, c), lambda i, j: (i, j, 0, 0)),
        out_shape=jax.ShapeDtypeStruct((b, h, w, c), jnp.float32),
    )(conv)


_CH = 8
_SUP = 8


def _sc_gather_body(table_h, idx_h, wts_h, flg_h, skip_h, fl_v, idx_v, wts_v,
                    rows_v, out_v, gsem, *, nn, c):
    nw = 32
    wid = lax.axis_index("s") * 2 + lax.axis_index("c")
    nsup = nn // (_CH * _SUP)
    q, r = divmod(nsup, nw)
    start = wid * q + jnp.minimum(wid, r)
    count = q + jnp.where(wid < r, 1, 0)
    cc = c // 16

    pltpu.sync_copy(flg_h.at[pl.ds(start * _SUP, q * _SUP)],
                    fl_v.at[pl.ds(0, q * _SUP)])

    @pl.when(wid < r)
    def _():
        pltpu.sync_copy(flg_h.at[pl.ds((start + q) * _SUP, _SUP)],
                        fl_v.at[pl.ds(q * _SUP, _SUP)])

    def sup(j, carry):
        sid = start + j

        def chunk(c8, carry2):
            cid = sid * _SUP + c8
            base = cid * _CH
            flr = fl_v[j * _SUP + c8, :]
            chunk_nz = flr[0] > 0.0

            @pl.when(chunk_nz)
            def _():
                pltpu.sync_copy(idx_h.at[pl.ds(cid * 128, 128)], idx_v)
                gcp = pltpu.async_copy(table_h.at[idx_v], rows_v, gsem)
                pltpu.sync_copy(wts_h.at[pl.ds(base, _CH)], wts_v)
                gcp.wait()
                for i in range(_CH):
                    node_nz = flr[1 + i] > 0.0

                    @pl.when(node_nz)
                    def _(i=i):
                        wrow = wts_v[i, :]

                        def rstep(rr, accs):
                            w_spl = wrow.at[jnp.full((16,), rr, jnp.int32)] \
                                .get(mode="promise_in_bounds")
                            new = []
                            for ch in range(cc):
                                v = rows_v[i * 16 + rr, pl.ds(ch * 16, 16)]
                                new.append(accs[ch] + v * w_spl)
                            return tuple(new)

                        accs = lax.fori_loop(
                            0, 16, rstep,
                            tuple(jnp.zeros((16,), jnp.float32)
                                  for _ in range(cc)))
                        for ch in range(cc):
                            out_v[i, pl.ds(ch * 16, 16)] = accs[ch]

                    @pl.when(jnp.logical_not(node_nz))
                    def _(i=i):
                        for ch in range(cc):
                            out_v[i, pl.ds(ch * 16, 16)] = jnp.zeros(
                                (16,), jnp.float32)

                pltpu.sync_copy(out_v, skip_h.at[pl.ds(base, _CH)])

            return carry2

        lax.fori_loop(0, _SUP, chunk, 0)
        return carry

    lax.fori_loop(0, count, sup, 0)


def _sc_gather(table, idx, wts, flags, nn, c):
    kfn = functools.partial(
        pl.kernel,
        out_type=jax.ShapeDtypeStruct((nn, c), jnp.float32),
        mesh=plsc.VectorSubcoreMesh(core_axis_name="c", subcore_axis_name="s",
                                    num_cores=2, num_subcores=16),
        scratch_types=[
            pltpu.VMEM(((nn // (_CH * _SUP * 32) + 1) * _SUP, 16),
                       jnp.float32),
            pltpu.VMEM((128,), jnp.int32),
            pltpu.VMEM((_CH, 16), jnp.float32),
            pltpu.VMEM((_CH * 16, c), jnp.float32),
            pltpu.VMEM((_CH, c), jnp.float32),
            pltpu.SemaphoreType.DMA,
        ],
        compiler_params=pltpu.CompilerParams(use_tc_tiling_on_sc=True),
    )(functools.partial(_sc_gather_body, nn=nn, c=c))
    return kfn(table, idx, wts, flags)


def _asm_body(x_ref, skip_ref, pos_ref, wts_ref, out_ref, *, c):
    out_ref[:, 0:c] = x_ref[...]
    sel = jnp.sum(jnp.abs(wts_ref[...]), axis=1, keepdims=True) > 0.0
    out_ref[:, c:2 * c] = jnp.where(sel, skip_ref[...], 0.0)
    out_ref[:, 2 * c:2 * c + 2] = pos_ref[...]


def _asm(xf, skip, pos, wts, bn=2000):
    nn, c = xf.shape
    grid = nn // bn
    return pl.pallas_call(
        functools.partial(_asm_body, c=c),
        grid=(grid,),
        in_specs=[
            pl.BlockSpec((bn, c), lambda i: (i, 0)),
            pl.BlockSpec((bn, c), lambda i: (i, 0)),
            pl.BlockSpec((bn, 2), lambda i: (i, 0)),
            pl.BlockSpec((bn, 16), lambda i: (i, 0)),
        ],
        out_specs=pl.BlockSpec((bn, 2 * c + 2), lambda i: (i, 0)),
        out_shape=jax.ShapeDtypeStruct((nn, 2 * c + 2), jnp.float32),
    )(xf, skip, pos, wts)


def kernel(x, adj, conv_layer, W):
    B, N, C = x.shape
    _, _, H, Wd = conv_layer.shape
    S = Wd
    nn = B * N
    xf = x.reshape(nn, C)
    pos_f, idx, wts, flags = _prep(xf, W, n_per_b=N, S=S)
    table = _transpose_table(conv_layer).reshape(B * H * Wd, C)
    skip = _sc_gather(table, idx.reshape(nn * 16), wts, flags, nn, C)
    sel = (jnp.sum(jnp.abs(wts), axis=1, keepdims=True) > 0.0).reshape(B, N, 1)
    pos = pos_f.reshape(B, N, 2)
    out = jnp.concatenate(
        [x, jnp.where(sel, skip.reshape(B, N, C), 0.0), pos], axis=-1)
    return (out, pos)


# speedup vs baseline: 1.3433x; 1.0000x over previous
import functools

import jax
import jax.numpy as jnp
from jax import lax
from jax.experimental import pallas as pl
from jax.experimental.pallas import tpu as pltpu
import jax.experimental.pallas.tpu_sc as plsc


def _buckets(p, S):
    f = jnp.floor(p)
    r0 = jnp.clip(f - 1.0, 0.0, S - 4.0)
    start = (p - 1.0) - 0.5
    ylfs, yhfs, lys, hys, vs = [], [], [], [], []
    for j in range(3):
        q = start + (j + 0.5)
        v = (q > -1.0) & (q < S)
        y = jnp.maximum(q, 0.0)
        ylf = jnp.floor(y)
        over = ylf >= S - 1.0
        ylf = jnp.where(over, S - 1.0, ylf)
        yhf = jnp.where(over, S - 1.0, ylf + 1.0)
        ly = jnp.where(over, 0.0, y - ylf)
        hy = 1.0 - ly
        ylfs.append(ylf); yhfs.append(yhf); lys.append(ly); hys.append(hy)
        vs.append(v.astype(p.dtype))
    cols = []
    for k in range(4):
        r = r0 + k
        acc = jnp.zeros_like(p)
        for j in range(3):
            acc = acc + vs[j] * (hys[j] * (ylfs[j] == r).astype(p.dtype)
                                 + lys[j] * (yhfs[j] == r).astype(p.dtype))
        cols.append(acc)
    return r0, jnp.concatenate(cols, axis=1)


def _prep_body(x_ref, w_ref, pos_ref, idx_ref, wts_ref, flg_ref, *,
               bn, n_per_b, S):
    i = pl.program_id(0)
    x = x_ref[...]
    w = w_ref[...]
    pos = jnp.dot(x, w, preferred_element_type=jnp.float32)
    pos_ref[...] = pos
    p = pos * float(S)
    px = p[:, 0:1]
    py = p[:, 1:2]
    cx0, wx = _buckets(px, S)
    ry0, wy = _buckets(py, S)
    rows = lax.broadcasted_iota(jnp.int32, (bn, 1), 0) + i * bn
    bvec = (rows // n_per_b).astype(jnp.float32)
    base = (bvec * float(S * S) + ry0 * float(S) + cx0).astype(jnp.int32)
    t = lax.broadcasted_iota(jnp.int32, (1, 16), 1)
    offs = (t // 4) * S + (t % 4)
    idx_ref[...] = base + offs
    wts = jnp.concatenate([wy[:, a:a + 1] * wx for a in range(4)], axis=1)
    wts_ref[...] = wts * (1.0 / 9.0)
    aw3 = jnp.sum(jnp.abs(wts).reshape(bn // 8, 8, 16), axis=2)
    nz8 = (aw3 > 0.0).astype(jnp.float32)
    any8 = (jnp.sum(aw3, axis=1, keepdims=True) > 0.0).astype(jnp.float32)
    flg_ref[...] = jnp.concatenate(
        [any8, nz8, jnp.zeros((bn // 8, 7), jnp.float32)], axis=1)


def _prep(xf, W, n_per_b, S, bn=1600):
    nn = xf.shape[0]
    c = xf.shape[1]
    grid = nn // bn
    return pl.pallas_call(
        functools.partial(_prep_body, bn=bn, n_per_b=n_per_b, S=S),
        grid=(grid,),
        in_specs=[
            pl.BlockSpec((bn, c), lambda i: (i, 0)),
            pl.BlockSpec((c, 2), lambda i: (0, 0)),
        ],
        out_specs=[
            pl.BlockSpec((bn, 2), lambda i: (i, 0)),
            pl.BlockSpec((bn, 16), lambda i: (i, 0)),
            pl.BlockSpec((bn, 16), lambda i: (i, 0)),
            pl.BlockSpec((bn // 8, 16), lambda i: (i, 0)),
        ],
        out_shape=[
            jax.ShapeDtypeStruct((nn, 2), jnp.float32),
            jax.ShapeDtypeStruct((nn, 16), jnp.int32),
            jax.ShapeDtypeStruct((nn, 16), jnp.float32),
            jax.ShapeDtypeStruct((nn // 8, 16), jnp.float32),
        ],
    )(xf, W)


def _tr_body(f_ref, t_ref):
    c = f_ref.shape[1]
    ii = lax.broadcasted_iota(jnp.int32, (c, c), 0)
    jj = lax.broadcasted_iota(jnp.int32, (c, c), 1)
    eye = (ii == jj).astype(jnp.float32)
    for yy in range(f_ref.shape[2]):
        x = f_ref[0, :, yy, :]
        t_ref[0, yy] = lax.dot_general(
            x, eye, (((0,), (0,)), ((), ())),
            preferred_element_type=jnp.float32)


def _transpose_table(conv, hblk=8):
    b, c, h, w = conv.shape
    return pl.pallas_call(
        _tr_body,
        grid=(b, h // hblk),
        in_specs=[pl.BlockSpec((1, c, hblk, w), lambda i, j: (i, 0, j, 0))],
        out_specs=pl.BlockSpec((1, hblk, w, c), lambda i, j: (i, j, 0, 0)),
        out_shape=jax.ShapeDtypeStruct((b, h, w, c), jnp.float32),
    )(conv)


_CH = 8
_SUP = 8


def _sc_gather_body(table_h, idx_h, wts_h, flg_h, skip_h, fl_v, idx_v, wts_v,
                    rows_v, out_v, gsem, *, nn, c):
    nw = 32
    wid = lax.axis_index("s") * 2 + lax.axis_index("c")
    nsup = nn // (_CH * _SUP)
    q, r = divmod(nsup, nw)
    start = wid * q + jnp.minimum(wid, r)
    count = q + jnp.where(wid < r, 1, 0)
    cc = c // 16

    pltpu.sync_copy(flg_h.at[pl.ds(start * _SUP, q * _SUP)],
                    fl_v.at[pl.ds(0, q * _SUP)])

    @pl.when(wid < r)
    def _():
        pltpu.sync_copy(flg_h.at[pl.ds((start + q) * _SUP, _SUP)],
                        fl_v.at[pl.ds(q * _SUP, _SUP)])

    def sup(j, carry):
        sid = start + j

        def chunk(c8, carry2):
            cid = sid * _SUP + c8
            base = cid * _CH
            flr = fl_v[j * _SUP + c8, :]
            chunk_nz = flr[0] > 0.0

            @pl.when(chunk_nz)
            def _():
                pltpu.sync_copy(idx_h.at[pl.ds(cid * 128, 128)], idx_v)
                gcp = pltpu.async_copy(table_h.at[idx_v], rows_v, gsem)
                pltpu.sync_copy(wts_h.at[pl.ds(base, _CH)], wts_v)
                gcp.wait()
                for i in range(_CH):
                    node_nz = flr[1 + i] > 0.0

                    @pl.when(node_nz)
                    def _(i=i):
                        wrow = wts_v[i, :]

                        def rstep(rr, accs):
                            w_spl = wrow.at[jnp.full((16,), rr, jnp.int32)] \
                                .get(mode="promise_in_bounds")
                            new = []
                            for ch in range(cc):
                                v = rows_v[i * 16 + rr, pl.ds(ch * 16, 16)]
                                new.append(accs[ch] + v * w_spl)
                            return tuple(new)

                        accs = lax.fori_loop(
                            0, 16, rstep,
                            tuple(jnp.zeros((16,), jnp.float32)
                                  for _ in range(cc)))
                        for ch in range(cc):
                            out_v[i, pl.ds(ch * 16, 16)] = accs[ch]

                    @pl.when(jnp.logical_not(node_nz))
                    def _(i=i):
                        for ch in range(cc):
                            out_v[i, pl.ds(ch * 16, 16)] = jnp.zeros(
                                (16,), jnp.float32)

                pltpu.sync_copy(out_v, skip_h.at[pl.ds(base, _CH)])

            return carry2

        lax.fori_loop(0, _SUP, chunk, 0)
        return carry

    lax.fori_loop(0, count, sup, 0)


def _sc_gather(table, idx, wts, flags, nn, c):
    kfn = functools.partial(
        pl.kernel,
        out_type=jax.ShapeDtypeStruct((nn, c), jnp.float32),
        mesh=plsc.VectorSubcoreMesh(core_axis_name="c", subcore_axis_name="s",
                                    num_cores=2, num_subcores=16),
        scratch_types=[
            pltpu.VMEM(((nn // (_CH * _SUP * 32) + 1) * _SUP, 16),
                       jnp.float32),
            pltpu.VMEM((128,), jnp.int32),
            pltpu.VMEM((_CH, 16), jnp.float32),
            pltpu.VMEM((_CH * 16, c), jnp.float32),
            pltpu.VMEM((_CH, c), jnp.float32),
            pltpu.SemaphoreType.DMA,
        ],
        compiler_params=pltpu.CompilerParams(use_tc_tiling_on_sc=True),
    )(functools.partial(_sc_gather_body, nn=nn, c=c))
    return kfn(table, idx, wts, flags)


def kernel(x, adj, conv_layer, W):
    B, N, C = x.shape
    _, _, H, Wd = conv_layer.shape
    S = Wd
    nn = B * N
    xf = x.reshape(nn, C)
    pos_f, idx, wts, flags = _prep(xf, W, n_per_b=N, S=S)
    table = _transpose_table(conv_layer).reshape(B * H * Wd, C)
    skip = _sc_gather(table, idx.reshape(nn * 16), wts, flags, nn, C)
    sel = (jnp.sum(jnp.abs(wts), axis=1, keepdims=True) > 0.0).reshape(B, N, 1)
    pos = pos_f.reshape(B, N, 2)
    out = jnp.concatenate(
        [x, jnp.where(sel, skip.reshape(B, N, C), 0.0), pos], axis=-1)
    return (out, pos)
